# Initial kernel scaffold; baseline (speedup 1.0000x reference)
import functools

import jax
import jax.numpy as jnp
from jax import lax
from jax.experimental import pallas as pl
from jax.experimental.pallas import tpu as pltpu
from jax.experimental.pallas import tpu_sc as plsc

VOCAB = 1000000
D_EMB = 64
D_MODEL = 128
B, S = 1024, 200
N_TOK = B * S

_INFO = plsc.get_sparse_core_info()
NC, NS = _INFO.num_cores, _INFO.num_subcores
NW = NC * NS
ROWS_PER_W = N_TOK // NW
CHUNK = 128
N_CHUNKS = ROWS_PER_W // CHUNK


def _gather_body(idx_hbm, table_hbm, out_hbm, idx_v, rows_v, sem):
    wid = lax.axis_index("s") * NC + lax.axis_index("c")
    base = wid * ROWS_PER_W
    pltpu.sync_copy(idx_hbm.at[pl.ds(base, ROWS_PER_W)],
                    idx_v.at[:].reshape(ROWS_PER_W))

    def step(j, _):
        pltpu.async_copy(table_hbm.at[idx_v.at[j]], rows_v, sem).wait()
        pltpu.sync_copy(rows_v, out_hbm.at[pl.ds(base + j * CHUNK, CHUNK)])
        return 0

    lax.fori_loop(0, N_CHUNKS, step, 0)


def _sc_gather(idx_flat, token_table):
    mesh = plsc.VectorSubcoreMesh(core_axis_name="c", subcore_axis_name="s")
    k = pl.kernel(
        _gather_body,
        mesh=mesh,
        out_type=jax.ShapeDtypeStruct((N_TOK, D_EMB), jnp.float32),
        scratch_types=[
            pltpu.VMEM((N_CHUNKS, CHUNK), jnp.int32),
            pltpu.VMEM((CHUNK, D_EMB), jnp.float32),
            pltpu.SemaphoreType.DMA,
        ],
    )
    return k(idx_flat, token_table)


def _tc_body(tok_ref, W_ref, b_ref, pos_ref, pos_W_ref, pos_b_ref,
             seg_ref, seg_W_ref, seg_b_ref, gamma_ref, beta_ref, out_ref):
    c = (jnp.dot(pos_ref[:], pos_W_ref[:],
                 preferred_element_type=jnp.float32)
         + pos_b_ref[:][None, :]
         + b_ref[:][None, :]
         + jnp.dot(seg_ref[:], seg_W_ref[:],
                   preferred_element_type=jnp.float32)
         + seg_b_ref[:][None, :])
    tok = tok_ref[:]
    bb = tok.shape[0]
    y = jnp.dot(tok.reshape(bb * S, D_EMB), W_ref[:],
                preferred_element_type=jnp.float32)
    y = y.reshape(bb, S, D_MODEL) + c[None, :, :]
    mu = jnp.mean(y, axis=-1, keepdims=True)
    d = y - mu
    var = jnp.mean(d * d, axis=-1, keepdims=True)
    out_ref[:] = d * lax.rsqrt(var + 1e-5) * gamma_ref[:] + beta_ref[:]


def _tc_compute(tok, W, b, pos_seq, pos_W, pos_b, seg_row, seg_W, seg_b,
                gamma, beta):
    BB = 64
    grid = (B // BB,)
    rep2 = lambda shape: pl.BlockSpec(shape, lambda i: (0, 0))
    rep1 = lambda shape: pl.BlockSpec(shape, lambda i: (0,))
    return pl.pallas_call(
        _tc_body,
        grid=grid,
        in_specs=[
            pl.BlockSpec((BB, S, D_EMB), lambda i: (i, 0, 0)),
            rep2((D_EMB, D_MODEL)),
            rep1((D_MODEL,)),
            rep2((S, D_EMB)),
            rep2((D_EMB, D_MODEL)),
            rep1((D_MODEL,)),
            rep2((1, D_EMB)),
            rep2((D_EMB, D_MODEL)),
            rep1((D_MODEL,)),
            rep1((D_MODEL,)),
            rep1((D_MODEL,)),
        ],
        out_specs=pl.BlockSpec((BB, S, D_MODEL), lambda i: (i, 0, 0)),
        out_shape=jax.ShapeDtypeStruct((B, S, D_MODEL), jnp.float32),
    )(tok, W, b, pos_seq, pos_W, pos_b, seg_row, seg_W, seg_b, gamma, beta)


def kernel(token_table, W, b, pos_table, pos_W, pos_b, seg_table, seg_W,
           seg_b, gamma, beta, sequence):
    idx_flat = sequence.reshape(N_TOK).astype(jnp.int32)
    tok = _sc_gather(idx_flat, token_table)
    tok = tok.reshape(B, S, D_EMB)
    return _tc_compute(tok, W, b, pos_table[:S], pos_W, pos_b,
                       seg_table[0:1], seg_W, seg_b, gamma, beta)


# speedup vs baseline: 1.5685x; 1.5685x over previous
import functools

import jax
import jax.numpy as jnp
from jax import lax
from jax.experimental import pallas as pl
from jax.experimental.pallas import tpu as pltpu
from jax.experimental.pallas import tpu_sc as plsc

VOCAB = 1000000
D_EMB = 64
D_MODEL = 128
B, S = 1024, 200
N_TOK = B * S

_INFO = plsc.get_sparse_core_info()
NC, NS = _INFO.num_cores, _INFO.num_subcores
NW = NC * NS
ROWS_PER_W = N_TOK // NW
CHUNK = 128
N_CHUNKS = ROWS_PER_W // CHUNK


def _gather_body(idx_hbm, table_hbm, out_hbm, idx_v, rows_v, sem):
    wid = lax.axis_index("s") * NC + lax.axis_index("c")
    base = wid * ROWS_PER_W
    pltpu.sync_copy(idx_hbm.at[wid], idx_v)

    def step(j, _):
        pltpu.async_copy(table_hbm.at[idx_v.at[j]], rows_v, sem).wait()
        pltpu.sync_copy(rows_v, out_hbm.at[pl.ds(base + j * CHUNK, CHUNK)])
        return 0

    lax.fori_loop(0, N_CHUNKS, step, 0)


def _sc_gather(idx3, token_table):
    mesh = plsc.VectorSubcoreMesh(core_axis_name="c", subcore_axis_name="s")
    k = pl.kernel(
        _gather_body,
        mesh=mesh,
        out_type=jax.ShapeDtypeStruct((N_TOK, D_EMB), jnp.float32),
        scratch_types=[
            pltpu.VMEM((N_CHUNKS, CHUNK), jnp.int32),
            pltpu.VMEM((CHUNK, D_EMB), jnp.float32),
            pltpu.SemaphoreType.DMA,
        ],
        compiler_params=pltpu.CompilerParams(use_tc_tiling_on_sc=False),
    )
    return k(idx3, token_table)


def _tc_body(tok_ref, W_ref, b_ref, pos_ref, pos_W_ref, pos_b_ref,
             seg_ref, seg_W_ref, seg_b_ref, gamma_ref, beta_ref, out_ref):
    c = (jnp.dot(pos_ref[:], pos_W_ref[:],
                 preferred_element_type=jnp.float32)
         + pos_b_ref[:][None, :]
         + b_ref[:][None, :]
         + jnp.dot(seg_ref[:], seg_W_ref[:],
                   preferred_element_type=jnp.float32)
         + seg_b_ref[:][None, :])
    tok = tok_ref[:]
    bb = tok.shape[0]
    y = jnp.dot(tok.reshape(bb * S, D_EMB), W_ref[:],
                preferred_element_type=jnp.float32)
    y = y.reshape(bb, S, D_MODEL) + c[None, :, :]
    mu = jnp.mean(y, axis=-1, keepdims=True)
    d = y - mu
    var = jnp.mean(d * d, axis=-1, keepdims=True)
    out_ref[:] = d * lax.rsqrt(var + 1e-5) * gamma_ref[:] + beta_ref[:]


def _tc_compute(tok, W, b, pos_seq, pos_W, pos_b, seg_row, seg_W, seg_b,
                gamma, beta):
    BB = 64
    grid = (B // BB,)
    rep2 = lambda shape: pl.BlockSpec(shape, lambda i: (0, 0))
    rep1 = lambda shape: pl.BlockSpec(shape, lambda i: (0,))
    return pl.pallas_call(
        _tc_body,
        grid=grid,
        in_specs=[
            pl.BlockSpec((BB, S, D_EMB), lambda i: (i, 0, 0)),
            rep2((D_EMB, D_MODEL)),
            rep1((D_MODEL,)),
            rep2((S, D_EMB)),
            rep2((D_EMB, D_MODEL)),
            rep1((D_MODEL,)),
            rep2((1, D_EMB)),
            rep2((D_EMB, D_MODEL)),
            rep1((D_MODEL,)),
            rep1((D_MODEL,)),
            rep1((D_MODEL,)),
        ],
        out_specs=pl.BlockSpec((BB, S, D_MODEL), lambda i: (i, 0, 0)),
        out_shape=jax.ShapeDtypeStruct((B, S, D_MODEL), jnp.float32),
    )(tok, W, b, pos_seq, pos_W, pos_b, seg_row, seg_W, seg_b, gamma, beta)


def kernel(token_table, W, b, pos_table, pos_W, pos_b, seg_table, seg_W,
           seg_b, gamma, beta, sequence):
    idx3 = sequence.astype(jnp.int32).reshape(NW, N_CHUNKS, CHUNK)
    tok = _sc_gather(idx3, token_table)
    tok = tok.reshape(B, S, D_EMB)
    return _tc_compute(tok, W, b, pos_table[:S], pos_W, pos_b,
                       seg_table[0:1], seg_W, seg_b, gamma, beta)
